# baseline (speedup 1.0000x reference)
import functools

import jax
import jax.numpy as jnp
from jax.experimental import pallas as pl
from jax.experimental.pallas import tpu as pltpu

_INTERPRET = False
_NSEM = 4
_LAG_ITERS = 8


def _gate_body(x_ref, wg_ref, bg_ref, idx_ref):
    logits = jnp.dot(x_ref[...], wg_ref[...], preferred_element_type=jnp.float32)
    logits = logits + bg_ref[...][None, :]
    m = jnp.max(logits, axis=-1, keepdims=True)
    p = jnp.exp(logits - m)
    p = p / jnp.sum(p, axis=-1, keepdims=True)
    pm = jnp.max(p, axis=-1, keepdims=True)
    lanes = jax.lax.broadcasted_iota(jnp.int32, p.shape, 1)
    idx_ref[...] = jnp.max(jnp.where(p >= pm, lanes, -1), axis=-1, keepdims=True)


def _dispatch_body(N, E, idx_ref, x_ref, we_ref, be_ref, out_ref, acc_ref,
                   *sems):
    xx = x_ref[...]
    for e in range(E):
        acc_ref[e] = (
            jnp.dot(xx, we_ref[e], preferred_element_type=jnp.float32)
            + be_ref[e][None, :]
        )

    def issue(i, _):
        for k in range(_NSEM):
            tok = i * _NSEM + k
            e = idx_ref[tok]
            pltpu.make_async_copy(acc_ref.at[e], out_ref.at[tok], sems[k]).start()

        @pl.when(i >= _LAG_ITERS)
        def _():
            for k in range(_NSEM):
                pltpu.make_async_copy(acc_ref.at[0], out_ref.at[0], sems[k]).wait()

        return 0

    jax.lax.fori_loop(0, N // _NSEM, issue, 0)
    for _ in range(_LAG_ITERS):
        for k in range(_NSEM):
            pltpu.make_async_copy(acc_ref.at[0], out_ref.at[0], sems[k]).wait()


def kernel(x, W_gate, b_gate, W_experts, b_experts):
    N, D_MODEL = x.shape
    E = W_gate.shape[1]
    D_FF = W_experts.shape[2]

    idx = pl.pallas_call(
        _gate_body,
        out_shape=jax.ShapeDtypeStruct((N, 1), jnp.int32),
        interpret=_INTERPRET,
    )(x, W_gate, b_gate)

    out = pl.pallas_call(
        functools.partial(_dispatch_body, N, E),
        in_specs=[
            pl.BlockSpec(memory_space=pltpu.SMEM),
            pl.BlockSpec(memory_space=pltpu.VMEM),
            pl.BlockSpec(memory_space=pltpu.VMEM),
            pl.BlockSpec(memory_space=pltpu.VMEM),
        ],
        out_specs=pl.BlockSpec(memory_space=pl.ANY),
        out_shape=jax.ShapeDtypeStruct((N, N, D_FF), jnp.float32),
        scratch_shapes=[pltpu.VMEM((E, N, D_FF), jnp.float32)]
        + [pltpu.SemaphoreType.DMA] * _NSEM,
        compiler_params=pltpu.CompilerParams(
            vmem_limit_bytes=128 * 1024 * 1024,
        ),
        interpret=_INTERPRET,
    )(idx.reshape(N), x, W_experts, b_experts)
    return out


# speedup vs baseline: 1.0012x; 1.0012x over previous
import functools

import jax
import jax.numpy as jnp
from jax.experimental import pallas as pl
from jax.experimental.pallas import tpu as pltpu

_INTERPRET = False
_LAG = 32


def _body(N, E, x_ref, wg_ref, bg_ref, we_ref, be_ref, out_ref,
          ord_vmem, ord_smem, acc_ref, sem, gsem):
    logits = jnp.dot(x_ref[...], wg_ref[...], preferred_element_type=jnp.float32)
    logits = logits + bg_ref[...][None, :]
    m = jnp.max(logits, axis=-1, keepdims=True)
    p = jnp.exp(logits - m)
    p = p / jnp.sum(p, axis=-1, keepdims=True)
    pm = jnp.max(p, axis=-1, keepdims=True)
    lanes = jax.lax.broadcasted_iota(jnp.int32, p.shape, 1)
    idx = jnp.max(jnp.where(p >= pm, lanes, -1), axis=-1, keepdims=True)

    oh = (lanes == idx).astype(jnp.float32)
    row_i = jax.lax.broadcasted_iota(jnp.int32, (N, N), 0)
    col_i = jax.lax.broadcasted_iota(jnp.int32, (N, N), 1)
    tril = (row_i >= col_i).astype(jnp.float32)
    cum_oh = jnp.dot(tril, oh, preferred_element_type=jnp.float32)
    counts = jnp.sum(oh, axis=0, keepdims=True)
    er = jax.lax.broadcasted_iota(jnp.int32, (E, E), 0)
    ec = jax.lax.broadcasted_iota(jnp.int32, (E, E), 1)
    ut = (er < ec).astype(jnp.float32)
    offs = jnp.dot(counts, ut, preferred_element_type=jnp.float32)
    pos = jnp.sum(oh * (offs + cum_oh - 1.0), axis=1, keepdims=True)
    perm = (pos == col_i.astype(jnp.float32)).astype(jnp.float32)
    ivec = jax.lax.broadcasted_iota(jnp.int32, (N, 1), 0).astype(jnp.float32)
    order = jax.lax.dot_general(
        perm, ivec, (((0,), (0,)), ((), ())),
        preferred_element_type=jnp.float32)

    ones = (col_i[:1, :] >= 0).astype(jnp.float32)
    counts_col = jax.lax.dot_general(
        oh, ones, (((0,), (1,)), ((), ())),
        preferred_element_type=jnp.float32)[:, :1]
    lt = (er > ec).astype(jnp.float32)
    offs_col = jnp.dot(lt, counts_col, preferred_element_type=jnp.float32)
    n_row = jnp.full((1, 1), float(N), dtype=jnp.float32)
    packed = jnp.concatenate([order, offs_col, n_row], axis=0)
    ord_vmem[...] = packed.astype(jnp.int32)
    pltpu.make_async_copy(ord_vmem, ord_smem, gsem).start()
    pltpu.make_async_copy(ord_vmem, ord_smem, gsem).wait()

    xx = x_ref[...]
    for e in range(E):
        acc_ref[e] = (
            jnp.dot(xx, we_ref[e], preferred_element_type=jnp.float32)
            + be_ref[e][None, :]
        )

        def issue(s, _, e=e):
            tok = ord_smem[s, 0]
            pltpu.make_async_copy(acc_ref.at[e], out_ref.at[tok], sem).start()

            @pl.when(s >= _LAG)
            def _():
                pltpu.make_async_copy(acc_ref.at[0], out_ref.at[0], sem).wait()

            return 0

        jax.lax.fori_loop(ord_smem[N + e, 0], ord_smem[N + e + 1, 0], issue, 0)

    for _ in range(_LAG):
        pltpu.make_async_copy(acc_ref.at[0], out_ref.at[0], sem).wait()


def kernel(x, W_gate, b_gate, W_experts, b_experts):
    N, D_MODEL = x.shape
    E = W_gate.shape[1]
    D_FF = W_experts.shape[2]

    out = pl.pallas_call(
        functools.partial(_body, N, E),
        in_specs=[
            pl.BlockSpec(memory_space=pltpu.VMEM),
            pl.BlockSpec(memory_space=pltpu.VMEM),
            pl.BlockSpec(memory_space=pltpu.VMEM),
            pl.BlockSpec(memory_space=pltpu.VMEM),
            pl.BlockSpec(memory_space=pltpu.VMEM),
        ],
        out_specs=pl.BlockSpec(memory_space=pl.ANY),
        out_shape=jax.ShapeDtypeStruct((N, N, D_FF), jnp.float32),
        scratch_shapes=[
            pltpu.VMEM((N + E + 1, 1), jnp.int32),
            pltpu.SMEM((N + E + 1, 1), jnp.int32),
            pltpu.VMEM((E, N, D_FF), jnp.float32),
            pltpu.SemaphoreType.DMA,
            pltpu.SemaphoreType.DMA,
        ],
        compiler_params=pltpu.CompilerParams(
            vmem_limit_bytes=128 * 1024 * 1024,
        ),
        interpret=_INTERPRET,
    )(x, W_gate, b_gate, W_experts, b_experts)
    return out
